# baseline (speedup 1.0000x reference)
import jax
import jax.numpy as jnp
from jax.experimental import pallas as pl
from jax.experimental.pallas import tpu as pltpu
from jax.experimental.pallas import tpu_sc as plsc

BATCH = 16384
EMB = 128
HID = EMB // 2
NCHUNK = 2
CHUNK = BATCH // NCHUNK
GATHER_WINDOW = 128
BLK = 2048
ROWTILES = BLK // 128


def _sc_gather(user, item, user_table, item_table):
    mesh = plsc.VectorSubcoreMesh(core_axis_name="core", subcore_axis_name="subcore")
    out_type = (
        jax.ShapeDtypeStruct((CHUNK, EMB), jnp.float32),
        jax.ShapeDtypeStruct((CHUNK, EMB), jnp.float32),
    )

    @pl.kernel(out_type=out_type, mesh=mesh,
               scratch_types=[pltpu.SemaphoreType.DMA, pltpu.SemaphoreType.DMA])
    def gather_kernel(u_hbm, i_hbm, ut_hbm, it_hbm, uo_hbm, io_hbm, usem, isem):
        def body(ui_vmem, ii_vmem, uo_vmem, io_vmem):
            cu = pltpu.async_copy(ut_hbm.at[ui_vmem.at[0]], uo_vmem, usem)
            ci = pltpu.async_copy(it_hbm.at[ii_vmem.at[0]], io_vmem, isem)
            cu.wait()
            ci.wait()

        pltpu.emit_pipeline(
            body,
            grid=(CHUNK // GATHER_WINDOW,),
            in_specs=[
                pl.BlockSpec((1, GATHER_WINDOW), lambda i: (0, i)),
                pl.BlockSpec((1, GATHER_WINDOW), lambda i: (0, i)),
            ],
            out_specs=[
                pl.BlockSpec((GATHER_WINDOW, EMB), lambda i: (i, 0)),
                pl.BlockSpec((GATHER_WINDOW, EMB), lambda i: (i, 0)),
            ],
            core_axis_name=("core", "subcore"),
            dimension_semantics=(pltpu.PARALLEL,),
        )(u_hbm, i_hbm, uo_hbm, io_hbm)

    return gather_kernel(
        user.reshape(1, CHUNK), item.reshape(1, CHUNK), user_table, item_table
    )


def _dot_t(x, w):
    return jax.lax.dot_general(x, w, (((1,), (1,)), ((), ())),
                               preferred_element_type=jnp.float32)


def _mlp_body(ue_ref, ie_ref, w1_ref, b1_ref, w2_ref, b2_ref,
              w3_ref, b3_ref, o_ref):
    h = _dot_t(ue_ref[...], w1_ref[:, :EMB])
    h = h + _dot_t(ie_ref[...], w1_ref[:, EMB:])
    h = jnp.maximum(h + b1_ref[...], 0.0)
    h2 = jnp.maximum(_dot_t(h, w2_ref[...]) + b2_ref[...], 0.0)
    res = jnp.sum(h2 * w3_ref[...], axis=1) + b3_ref[0, 0]
    o_ref[...] = res.reshape(ROWTILES, 128)


def _tc_mlp(ue, ie, W1, b1, W2, b2, w3, b3):
    grid = (CHUNK // BLK,)
    out = pl.pallas_call(
        _mlp_body,
        grid=grid,
        in_specs=[
            pl.BlockSpec((BLK, EMB), lambda i: (i, 0)),
            pl.BlockSpec((BLK, EMB), lambda i: (i, 0)),
            pl.BlockSpec((EMB, 2 * EMB), lambda i: (0, 0)),
            pl.BlockSpec((1, EMB), lambda i: (0, 0)),
            pl.BlockSpec((HID, EMB), lambda i: (0, 0)),
            pl.BlockSpec((1, HID), lambda i: (0, 0)),
            pl.BlockSpec((1, HID), lambda i: (0, 0)),
            pl.BlockSpec((1, 1), lambda i: (0, 0)),
        ],
        out_specs=pl.BlockSpec((ROWTILES, 128), lambda i: (i, 0)),
        out_shape=jax.ShapeDtypeStruct((CHUNK // 128, 128), jnp.float32),
    )(ue, ie, W1, b1, W2, b2, w3, b3)
    return out.reshape(CHUNK)


def kernel(user, item, user_table, item_table, W1, b1, W2, b2, W3, b3):
    user = user.astype(jnp.int32)
    item = item.astype(jnp.int32)
    b1r = b1.reshape(1, EMB)
    b2r = b2.reshape(1, HID)
    w3r = W3.reshape(1, HID)
    b3r = b3.reshape(1, 1)
    embs = [
        _sc_gather(user[k * CHUNK:(k + 1) * CHUNK],
                   item[k * CHUNK:(k + 1) * CHUNK],
                   user_table, item_table)
        for k in range(NCHUNK)
    ]
    outs = [
        _tc_mlp(ue, ie, W1, b1r, W2, b2r, w3r, b3r)
        for (ue, ie) in embs
    ]
    return jnp.concatenate(outs, axis=0)


# speedup vs baseline: 1.0130x; 1.0130x over previous
import jax
import jax.numpy as jnp
from jax.experimental import pallas as pl
from jax.experimental.pallas import tpu as pltpu
from jax.experimental.pallas import tpu_sc as plsc

BATCH = 16384
EMB = 128
HID = EMB // 2
NCHUNK = 2
CHUNK = BATCH // NCHUNK
GATHER_WINDOW = 128
BLK = 2048
ROWTILES = BLK // 128


def _sc_gather(user2d, item2d, user_table, item_table, k):
    mesh = plsc.VectorSubcoreMesh(core_axis_name="core", subcore_axis_name="subcore")
    out_type = (
        jax.ShapeDtypeStruct((CHUNK, EMB), jnp.float32),
        jax.ShapeDtypeStruct((CHUNK, EMB), jnp.float32),
    )
    base = k * (CHUNK // GATHER_WINDOW)

    @pl.kernel(out_type=out_type, mesh=mesh,
               scratch_types=[pltpu.SemaphoreType.DMA, pltpu.SemaphoreType.DMA])
    def gather_kernel(u_hbm, i_hbm, ut_hbm, it_hbm, uo_hbm, io_hbm, usem, isem):
        def body(ui_vmem, ii_vmem, uo_vmem, io_vmem):
            cu = pltpu.async_copy(ut_hbm.at[ui_vmem.at[0]], uo_vmem, usem)
            ci = pltpu.async_copy(it_hbm.at[ii_vmem.at[0]], io_vmem, isem)
            cu.wait()
            ci.wait()

        pltpu.emit_pipeline(
            body,
            grid=(CHUNK // GATHER_WINDOW,),
            in_specs=[
                pl.BlockSpec((1, GATHER_WINDOW), lambda i: (0, base + i)),
                pl.BlockSpec((1, GATHER_WINDOW), lambda i: (0, base + i)),
            ],
            out_specs=[
                pl.BlockSpec((GATHER_WINDOW, EMB), lambda i: (i, 0)),
                pl.BlockSpec((GATHER_WINDOW, EMB), lambda i: (i, 0)),
            ],
            core_axis_name=("core", "subcore"),
            dimension_semantics=(pltpu.PARALLEL,),
        )(u_hbm, i_hbm, uo_hbm, io_hbm)

    return gather_kernel(user2d, item2d, user_table, item_table)


def _dot_t(x, w):
    return jax.lax.dot_general(x, w, (((1,), (1,)), ((), ())),
                               preferred_element_type=jnp.float32)


def _mlp_body(ue_ref, ie_ref, w1_ref, b1_ref, w2_ref, b2_ref,
              w3_ref, b3_ref, o_ref):
    h = _dot_t(ue_ref[...], w1_ref[:, :EMB])
    h = h + _dot_t(ie_ref[...], w1_ref[:, EMB:])
    h = jnp.maximum(h + b1_ref[...], 0.0)
    h2 = jnp.maximum(_dot_t(h, w2_ref[...]) + b2_ref[...], 0.0)
    res = jnp.sum(h2 * w3_ref[...], axis=1) + b3_ref[0, 0]
    o_ref[...] = res.reshape(ROWTILES, 128)


def _tc_mlp(ue, ie, W1, b1, W2, b2, w3, b3):
    grid = (CHUNK // BLK,)
    out = pl.pallas_call(
        _mlp_body,
        grid=grid,
        in_specs=[
            pl.BlockSpec((BLK, EMB), lambda i: (i, 0)),
            pl.BlockSpec((BLK, EMB), lambda i: (i, 0)),
            pl.BlockSpec((EMB, 2 * EMB), lambda i: (0, 0)),
            pl.BlockSpec((1, EMB), lambda i: (0, 0)),
            pl.BlockSpec((HID, EMB), lambda i: (0, 0)),
            pl.BlockSpec((1, HID), lambda i: (0, 0)),
            pl.BlockSpec((1, HID), lambda i: (0, 0)),
            pl.BlockSpec((1, 1), lambda i: (0, 0)),
        ],
        out_specs=pl.BlockSpec((ROWTILES, 128), lambda i: (i, 0)),
        out_shape=jax.ShapeDtypeStruct((CHUNK // 128, 128), jnp.float32),
    )(ue, ie, W1, b1, W2, b2, w3, b3)
    return out


def kernel(user, item, user_table, item_table, W1, b1, W2, b2, W3, b3):
    user2d = user.astype(jnp.int32).reshape(1, BATCH)
    item2d = item.astype(jnp.int32).reshape(1, BATCH)
    b1r = b1.reshape(1, EMB)
    b2r = b2.reshape(1, HID)
    w3r = W3.reshape(1, HID)
    b3r = b3.reshape(1, 1)
    embs = [
        _sc_gather(user2d, item2d, user_table, item_table, k)
        for k in range(NCHUNK)
    ]
    outs = [
        _tc_mlp(ue, ie, W1, b1r, W2, b2r, w3r, b3r)
        for (ue, ie) in embs
    ]
    return jnp.concatenate(outs, axis=0).reshape(BATCH)


# speedup vs baseline: 1.1472x; 1.1325x over previous
import jax
import jax.numpy as jnp
from jax.experimental import pallas as pl
from jax.experimental.pallas import tpu as pltpu
from jax.experimental.pallas import tpu_sc as plsc

BATCH = 16384
EMB = 128
HID = EMB // 2
NCHUNK = 1
CHUNK = BATCH // NCHUNK
GATHER_WINDOW = 128
BLK = 4096
ROWTILES = BLK // 128


def _sc_gather(user2d, item2d, user_table, item_table, k):
    mesh = plsc.VectorSubcoreMesh(core_axis_name="core", subcore_axis_name="subcore")
    out_type = (
        jax.ShapeDtypeStruct((CHUNK, EMB), jnp.float32),
        jax.ShapeDtypeStruct((CHUNK, EMB), jnp.float32),
    )
    base = k * (CHUNK // GATHER_WINDOW)

    @pl.kernel(out_type=out_type, mesh=mesh,
               scratch_types=[pltpu.SemaphoreType.DMA, pltpu.SemaphoreType.DMA])
    def gather_kernel(u_hbm, i_hbm, ut_hbm, it_hbm, uo_hbm, io_hbm, usem, isem):
        def body(ui_vmem, ii_vmem, uo_vmem, io_vmem):
            cu = pltpu.async_copy(ut_hbm.at[ui_vmem.at[0]], uo_vmem, usem)
            ci = pltpu.async_copy(it_hbm.at[ii_vmem.at[0]], io_vmem, isem)
            cu.wait()
            ci.wait()

        pltpu.emit_pipeline(
            body,
            grid=(CHUNK // GATHER_WINDOW,),
            in_specs=[
                pl.BlockSpec((1, GATHER_WINDOW), lambda i: (0, base + i)),
                pl.BlockSpec((1, GATHER_WINDOW), lambda i: (0, base + i)),
            ],
            out_specs=[
                pl.BlockSpec((GATHER_WINDOW, EMB), lambda i: (i, 0)),
                pl.BlockSpec((GATHER_WINDOW, EMB), lambda i: (i, 0)),
            ],
            core_axis_name=("core", "subcore"),
            dimension_semantics=(pltpu.PARALLEL,),
        )(u_hbm, i_hbm, uo_hbm, io_hbm)

    return gather_kernel(user2d, item2d, user_table, item_table)


def _dot_t(x, w):
    return jax.lax.dot_general(x, w, (((1,), (1,)), ((), ())),
                               preferred_element_type=jnp.float32)


def _mlp_body(ue_ref, ie_ref, w1_ref, b1_ref, w2_ref, b2_ref,
              w3_ref, b3_ref, o_ref):
    h = _dot_t(ue_ref[...], w1_ref[:, :EMB])
    h = h + _dot_t(ie_ref[...], w1_ref[:, EMB:])
    h = jnp.maximum(h + b1_ref[...], 0.0)
    h2 = jnp.maximum(_dot_t(h, w2_ref[...]) + b2_ref[...], 0.0)
    res = jnp.sum(h2 * w3_ref[...], axis=1) + b3_ref[0, 0]
    o_ref[...] = res.reshape(ROWTILES, 128)


def _tc_mlp(ue, ie, W1, b1, W2, b2, w3, b3):
    grid = (CHUNK // BLK,)
    out = pl.pallas_call(
        _mlp_body,
        grid=grid,
        in_specs=[
            pl.BlockSpec((BLK, EMB), lambda i: (i, 0)),
            pl.BlockSpec((BLK, EMB), lambda i: (i, 0)),
            pl.BlockSpec((EMB, 2 * EMB), lambda i: (0, 0)),
            pl.BlockSpec((1, EMB), lambda i: (0, 0)),
            pl.BlockSpec((HID, EMB), lambda i: (0, 0)),
            pl.BlockSpec((1, HID), lambda i: (0, 0)),
            pl.BlockSpec((1, HID), lambda i: (0, 0)),
            pl.BlockSpec((1, 1), lambda i: (0, 0)),
        ],
        out_specs=pl.BlockSpec((ROWTILES, 128), lambda i: (i, 0)),
        out_shape=jax.ShapeDtypeStruct((CHUNK // 128, 128), jnp.float32),
    )(ue, ie, W1, b1, W2, b2, w3, b3)
    return out


def kernel(user, item, user_table, item_table, W1, b1, W2, b2, W3, b3):
    user2d = user.astype(jnp.int32).reshape(1, BATCH)
    item2d = item.astype(jnp.int32).reshape(1, BATCH)
    b1r = b1.reshape(1, EMB)
    b2r = b2.reshape(1, HID)
    w3r = W3.reshape(1, HID)
    b3r = b3.reshape(1, 1)
    embs = [
        _sc_gather(user2d, item2d, user_table, item_table, k)
        for k in range(NCHUNK)
    ]
    outs = [
        _tc_mlp(ue, ie, W1, b1r, W2, b2r, w3r, b3r)
        for (ue, ie) in embs
    ]
    return jnp.concatenate(outs, axis=0).reshape(BATCH)


# speedup vs baseline: 1.1508x; 1.0031x over previous
import jax
import jax.numpy as jnp
from jax.experimental import pallas as pl
from jax.experimental.pallas import tpu as pltpu
from jax.experimental.pallas import tpu_sc as plsc

BATCH = 16384
EMB = 128
HID = EMB // 2
NCHUNK = 1
CHUNK = BATCH // NCHUNK
GATHER_WINDOW = 128
BLK = 8192
ROWTILES = BLK // 128


def _sc_gather(user2d, item2d, user_table, item_table, k):
    mesh = plsc.VectorSubcoreMesh(core_axis_name="core", subcore_axis_name="subcore")
    out_type = (
        jax.ShapeDtypeStruct((CHUNK, EMB), jnp.float32),
        jax.ShapeDtypeStruct((CHUNK, EMB), jnp.float32),
    )
    base = k * (CHUNK // GATHER_WINDOW)

    @pl.kernel(out_type=out_type, mesh=mesh,
               scratch_types=[pltpu.SemaphoreType.DMA, pltpu.SemaphoreType.DMA])
    def gather_kernel(u_hbm, i_hbm, ut_hbm, it_hbm, uo_hbm, io_hbm, usem, isem):
        def body(ui_vmem, ii_vmem, uo_vmem, io_vmem):
            cu = pltpu.async_copy(ut_hbm.at[ui_vmem.at[0]], uo_vmem, usem)
            ci = pltpu.async_copy(it_hbm.at[ii_vmem.at[0]], io_vmem, isem)
            cu.wait()
            ci.wait()

        pltpu.emit_pipeline(
            body,
            grid=(CHUNK // GATHER_WINDOW,),
            in_specs=[
                pl.BlockSpec((1, GATHER_WINDOW), lambda i: (0, base + i)),
                pl.BlockSpec((1, GATHER_WINDOW), lambda i: (0, base + i)),
            ],
            out_specs=[
                pl.BlockSpec((GATHER_WINDOW, EMB), lambda i: (i, 0)),
                pl.BlockSpec((GATHER_WINDOW, EMB), lambda i: (i, 0)),
            ],
            core_axis_name=("core", "subcore"),
            dimension_semantics=(pltpu.PARALLEL,),
        )(u_hbm, i_hbm, uo_hbm, io_hbm)

    return gather_kernel(user2d, item2d, user_table, item_table)


def _dot_t(x, w):
    return jax.lax.dot_general(x, w, (((1,), (1,)), ((), ())),
                               preferred_element_type=jnp.float32)


def _mlp_body(ue_ref, ie_ref, w1_ref, b1_ref, w2_ref, b2_ref,
              w3_ref, b3_ref, o_ref):
    h = _dot_t(ue_ref[...], w1_ref[:, :EMB])
    h = h + _dot_t(ie_ref[...], w1_ref[:, EMB:])
    h = jnp.maximum(h + b1_ref[...], 0.0)
    h2 = jnp.maximum(_dot_t(h, w2_ref[...]) + b2_ref[...], 0.0)
    res = jnp.sum(h2 * w3_ref[...], axis=1) + b3_ref[0, 0]
    o_ref[...] = res.reshape(ROWTILES, 128)


def _tc_mlp(ue, ie, W1, b1, W2, b2, w3, b3):
    grid = (CHUNK // BLK,)
    out = pl.pallas_call(
        _mlp_body,
        grid=grid,
        in_specs=[
            pl.BlockSpec((BLK, EMB), lambda i: (i, 0)),
            pl.BlockSpec((BLK, EMB), lambda i: (i, 0)),
            pl.BlockSpec((EMB, 2 * EMB), lambda i: (0, 0)),
            pl.BlockSpec((1, EMB), lambda i: (0, 0)),
            pl.BlockSpec((HID, EMB), lambda i: (0, 0)),
            pl.BlockSpec((1, HID), lambda i: (0, 0)),
            pl.BlockSpec((1, HID), lambda i: (0, 0)),
            pl.BlockSpec((1, 1), lambda i: (0, 0)),
        ],
        out_specs=pl.BlockSpec((ROWTILES, 128), lambda i: (i, 0)),
        out_shape=jax.ShapeDtypeStruct((CHUNK // 128, 128), jnp.float32),
    )(ue, ie, W1, b1, W2, b2, w3, b3)
    return out


def kernel(user, item, user_table, item_table, W1, b1, W2, b2, W3, b3):
    user2d = user.astype(jnp.int32).reshape(1, BATCH)
    item2d = item.astype(jnp.int32).reshape(1, BATCH)
    b1r = b1.reshape(1, EMB)
    b2r = b2.reshape(1, HID)
    w3r = W3.reshape(1, HID)
    b3r = b3.reshape(1, 1)
    embs = [
        _sc_gather(user2d, item2d, user_table, item_table, k)
        for k in range(NCHUNK)
    ]
    outs = [
        _tc_mlp(ue, ie, W1, b1r, W2, b2r, w3r, b3r)
        for (ue, ie) in embs
    ]
    return jnp.concatenate(outs, axis=0).reshape(BATCH)
